# baseline (speedup 1.0000x reference)
import jax
import jax.numpy as jnp
from jax import lax
from jax.experimental import pallas as pl
from jax.experimental.pallas import tpu as pltpu
from jax.experimental.pallas import tpu_sc as plsc

_N = 10000
_E = 320000
_D = 128
_B = 64

_NC = 2
_NS = 16
_NW = _NC * _NS
_EPT = _E // _NW
_CH = ((0, 128), (128, 72))
_NB = len(_CH)
_GK = 200
_GROUPS = _EPT // _GK
_NP = 10240
_RPT = _NP // _NS

_mesh = plsc.VectorSubcoreMesh(core_axis_name="c", subcore_axis_name="s")


def _seg_body(x_hbm, src_hbm, dst_hbm, zrow_hbm, part_hbm,
              acc, sxa, a0, a1, sxb, b0, b1, rows,
              semia, semib, sg0, sg1, sems):
    c = lax.axis_index("c")
    s = lax.axis_index("s")
    wid = c * _NS + s
    r0 = s * _RPT
    abufs = (a0, a1)
    bbufs = (b0, b1)
    sgs = (sg0, sg1)
    pltpu.sync_copy(zrow_hbm.at[pl.ds(r0, _RPT)], acc.at[pl.ds(r0, _RPT)])
    plsc.subcore_barrier()
    e0 = wid * _EPT

    def fire_idx(base, sx, dbufs, semi):
        pltpu.async_copy(src_hbm.at[pl.ds(base, _GK)], sx, semi)
        for b, (off, sz) in enumerate(_CH):
            pltpu.async_copy(dst_hbm.at[pl.ds(base + off, sz)], dbufs[b], semi)

    def wait_idx(sx, dbufs, semi):
        pltpu.make_async_copy(src_hbm.at[pl.ds(e0, _GK)], sx, semi).wait()
        for b, (off, sz) in enumerate(_CH):
            pltpu.make_async_copy(dst_hbm.at[pl.ds(e0, sz)], dbufs[b], semi).wait()

    def proc(sx, dbufs, semi):
        wait_idx(sx, dbufs, semi)
        gd = [pltpu.async_copy(x_hbm.at[sx.at[pl.ds(off, sz)]],
                               rows.at[pl.ds(off, sz)], sgs[b])
              for b, (off, sz) in enumerate(_CH)]
        sd = []
        for b, (off, sz) in enumerate(_CH):
            gd[b].wait()
            sd.append(pltpu.async_copy(rows.at[pl.ds(off, sz)],
                                       acc.at[dbufs[b]], sems, add=True))
        for d in sd:
            d.wait()

    fire_idx(e0, sxa, abufs, semia)

    def pair(t, carry):
        base_a = e0 + (2 * t) * _GK
        base_b = base_a + _GK
        base_n = jnp.where(t < _GROUPS // 2 - 1, base_a + 2 * _GK, e0)
        fire_idx(base_b, sxb, bbufs, semib)
        proc(sxa, abufs, semia)
        fire_idx(base_n, sxa, abufs, semia)
        proc(sxb, bbufs, semib)
        return carry

    lax.fori_loop(0, _GROUPS // 2, pair, 0)
    wait_idx(sxa, abufs, semia)
    plsc.subcore_barrier()
    pltpu.sync_copy(acc.at[pl.ds(r0, _RPT)], part_hbm.at[c, pl.ds(r0, _RPT)])


_seg_sum = pl.kernel(
    _seg_body,
    out_type=jax.ShapeDtypeStruct((_NC, _NP, _D), jnp.float32),
    mesh=_mesh,
    scratch_types=[pltpu.VMEM_SHARED((_NP, _D), jnp.float32)]
                  + [pltpu.VMEM((_GK,), jnp.int32)]
                  + [pltpu.VMEM((sz,), jnp.int32) for (_o, sz) in _CH]
                  + [pltpu.VMEM((_GK,), jnp.int32)]
                  + [pltpu.VMEM((sz,), jnp.int32) for (_o, sz) in _CH]
                  + [pltpu.VMEM((_GK, _D), jnp.float32)]
                  + [pltpu.SemaphoreType.DMA] * (3 + _NB),
)


def _deg_body(dst_hbm, zrow_hbm, onesr_hbm, part_hbm,
              acc, a0, a1, b0, b1, onesv, semia, semib, sems):
    c = lax.axis_index("c")
    s = lax.axis_index("s")
    wid = c * _NS + s
    r0 = s * _RPT
    abufs = (a0, a1)
    bbufs = (b0, b1)
    pltpu.sync_copy(zrow_hbm.at[pl.ds(r0, _RPT)], acc.at[pl.ds(r0, _RPT)])
    pltpu.sync_copy(onesr_hbm, onesv)
    plsc.subcore_barrier()
    e0 = wid * _EPT

    def fire_idx(base, dbufs, semi):
        for b, (off, sz) in enumerate(_CH):
            pltpu.async_copy(dst_hbm.at[pl.ds(base + off, sz)], dbufs[b], semi)

    def wait_idx(dbufs, semi):
        for b, (off, sz) in enumerate(_CH):
            pltpu.make_async_copy(dst_hbm.at[pl.ds(e0, sz)], dbufs[b], semi).wait()

    def proc(dbufs, semi):
        wait_idx(dbufs, semi)
        sd = [pltpu.async_copy(onesv.at[pl.ds(0, sz)], acc.at[dbufs[b]], sems, add=True)
              for b, (off, sz) in enumerate(_CH)]
        for d in sd:
            d.wait()

    fire_idx(e0, abufs, semia)

    def pair(t, carry):
        base_a = e0 + (2 * t) * _GK
        base_b = base_a + _GK
        base_n = jnp.where(t < _GROUPS // 2 - 1, base_a + 2 * _GK, e0)
        fire_idx(base_b, bbufs, semib)
        proc(abufs, semia)
        fire_idx(base_n, abufs, semia)
        proc(bbufs, semib)
        return carry

    lax.fori_loop(0, _GROUPS // 2, pair, 0)
    wait_idx(abufs, semia)
    plsc.subcore_barrier()
    pltpu.sync_copy(acc.at[pl.ds(r0, _RPT)], part_hbm.at[c, pl.ds(r0, _RPT)])


_deg_sum = pl.kernel(
    _deg_body,
    out_type=jax.ShapeDtypeStruct((_NC, _NP, _D), jnp.float32),
    mesh=_mesh,
    scratch_types=[pltpu.VMEM_SHARED((_NP, _D), jnp.float32)]
                  + [pltpu.VMEM((sz,), jnp.int32) for (_o, sz) in _CH]
                  + [pltpu.VMEM((sz,), jnp.int32) for (_o, sz) in _CH]
                  + [pltpu.VMEM((_CH[0][1], _D), jnp.float32)]
                  + [pltpu.SemaphoreType.DMA] * 3,
)


def _matmul_t(a, w):
    return lax.dot_general(a, w, (((1,), (1,)), ((), ())),
                           preferred_element_type=jnp.float32)


def _leaky(v, slope):
    return jnp.where(v > 0, v, slope * v)


def _graphnorm(h, ind, rcnt, gamma, beta, alpha):
    mean = lax.dot_general(ind, h, (((0,), (0,)), ((), ())),
                           preferred_element_type=jnp.float32) * rcnt
    sub = h - alpha * lax.dot_general(ind, mean, (((1,), (0,)), ((), ())),
                                      preferred_element_type=jnp.float32)
    var = lax.dot_general(ind, sub * sub, (((0,), (0,)), ((), ())),
                          preferred_element_type=jnp.float32) * rcnt
    inv = lax.rsqrt(var + 1e-5)
    return gamma * sub * lax.dot_general(ind, inv, (((1,), (0,)), ((), ())),
                                         preferred_element_type=jnp.float32) + beta


def _ind_rcnt(batch2d):
    ind = (lax.broadcasted_iota(jnp.int32, (_N, _B), 1) == batch2d)
    ind = ind.astype(jnp.float32)
    cnt = lax.dot_general(ind, jnp.ones((_N, 1), jnp.float32),
                          (((0,), (0,)), ((), ())),
                          preferred_element_type=jnp.float32)
    rcnt = 1.0 / jnp.maximum(cnt, 1.0)
    return ind, rcnt


def _tc1_body(x_ref, part_ref, degp_ref, batch_ref, wl_ref, bl_ref, wr_ref,
              g_ref, be_ref, a_ref, h1_ref):
    ssum = part_ref[0, :_N] + part_ref[1, :_N]
    deg = degp_ref[0, :_N] + degp_ref[1, :_N]
    agg = ssum / jnp.maximum(deg, 1.0)
    pre = _matmul_t(agg, wl_ref[...]) + bl_ref[...] + _matmul_t(x_ref[...], wr_ref[...])
    h = _leaky(pre, 0.01)
    ind, rcnt = _ind_rcnt(batch_ref[...])
    h1_ref[...] = _graphnorm(h, ind, rcnt, g_ref[...], be_ref[...], a_ref[...])


def _tc2_body(h1_ref, part_ref, degp_ref, batch_ref, wl_ref, bl_ref, wr_ref,
              g_ref, be_ref, a_ref, wc1_ref, bc1_ref, bng_ref, bnb_ref,
              bnm_ref, bnv_ref, wc2_ref, bc2_ref, out_ref):
    h1 = h1_ref[...]
    ssum = part_ref[0, :_N] + part_ref[1, :_N]
    deg = degp_ref[0, :_N] + degp_ref[1, :_N]
    agg = ssum / jnp.maximum(deg, 1.0)
    pre = _matmul_t(agg, wl_ref[...]) + bl_ref[...] + _matmul_t(h1, wr_ref[...])
    r = _leaky(pre, 0.01)
    ind, rcnt = _ind_rcnt(batch_ref[...])
    r = _graphnorm(r, ind, rcnt, g_ref[...], be_ref[...], a_ref[...])
    h = h1 + r
    p = lax.dot_general(ind, h, (((0,), (0,)), ((), ())),
                        preferred_element_type=jnp.float32) * rcnt
    c1 = _matmul_t(p, wc1_ref[...]) + bc1_ref[...]
    c1 = (c1 - bnm_ref[...]) * lax.rsqrt(bnv_ref[...] + 1e-5) * bng_ref[...] + bnb_ref[...]
    c1 = _leaky(c1, 0.1)
    c2 = _matmul_t(c1, wc2_ref[...]) + bc2_ref[...]
    m = jnp.max(c2, axis=1, keepdims=True)
    sh = c2 - m
    out_ref[...] = sh - jnp.log(jnp.sum(jnp.exp(sh), axis=1, keepdims=True))


_tc1 = pl.pallas_call(_tc1_body, out_shape=jax.ShapeDtypeStruct((_N, _D), jnp.float32))
_tc2 = pl.pallas_call(_tc2_body, out_shape=jax.ShapeDtypeStruct((_B, 2), jnp.float32))


def kernel(x, edge_index, batch, Wl1, bl1, Wr1, g1, be1, a1, Wl2, bl2, Wr2,
           g2, be2, a2, Wc1, bc1, bng, bnb, bnm, bnv, Wc2, bc2):
    src = edge_index[0]
    dst = edge_index[1]
    zrow = jnp.zeros((_NP, _D), jnp.float32)
    onesr = jnp.ones((_CH[0][1], _D), jnp.float32)
    batch2d = batch.reshape(_N, 1)

    part1 = _seg_sum(x, src, dst, zrow)
    degp = _deg_sum(dst, zrow, onesr)
    h1 = _tc1(x, part1, degp, batch2d, Wl1, bl1.reshape(1, _D), Wr1,
              g1.reshape(1, _D), be1.reshape(1, _D), a1.reshape(1, _D))
    part2 = _seg_sum(h1, src, dst, zrow)
    out = _tc2(h1, part2, degp, batch2d, Wl2, bl2.reshape(1, _D), Wr2,
               g2.reshape(1, _D), be2.reshape(1, _D), a2.reshape(1, _D),
               Wc1, bc1.reshape(1, _B), bng.reshape(1, _B), bnb.reshape(1, _B),
               bnm.reshape(1, _B), bnv.reshape(1, _B), Wc2, bc2.reshape(1, 2))
    return out


# speedup vs baseline: 1.0479x; 1.0479x over previous
import jax
import jax.numpy as jnp
from jax import lax
from jax.experimental import pallas as pl
from jax.experimental.pallas import tpu as pltpu
from jax.experimental.pallas import tpu_sc as plsc

_N = 10000
_E = 320000
_D = 128
_B = 64

_NC = 2
_NS = 16
_NW = _NC * _NS
_EPT = _E // _NW
_CH = tuple((40 * i, 40) for i in range(5))
_NB = len(_CH)
_GK = 200
_GROUPS = _EPT // _GK
_NP = 10240
_RPT = _NP // _NS

_mesh = plsc.VectorSubcoreMesh(core_axis_name="c", subcore_axis_name="s")


def _seg_body(x_hbm, src_hbm, dst_hbm, zrow_hbm, part_hbm, *scr):
    acc = scr[0]
    sxa, abufs = scr[1], scr[2:2 + _NB]
    sxb, bbufs = scr[2 + _NB], scr[3 + _NB:3 + 2 * _NB]
    rows = scr[3 + 2 * _NB]
    semia, semib = scr[4 + 2 * _NB], scr[5 + 2 * _NB]
    sgs = scr[6 + 2 * _NB:6 + 3 * _NB]
    sems = scr[6 + 3 * _NB]
    c = lax.axis_index("c")
    s = lax.axis_index("s")
    wid = c * _NS + s
    r0 = s * _RPT
    pltpu.sync_copy(zrow_hbm.at[pl.ds(r0, _RPT)], acc.at[pl.ds(r0, _RPT)])
    plsc.subcore_barrier()
    e0 = wid * _EPT

    def fire_idx(base, sx, dbufs, semi):
        pltpu.async_copy(src_hbm.at[pl.ds(base, _GK)], sx, semi)
        for b, (off, sz) in enumerate(_CH):
            pltpu.async_copy(dst_hbm.at[pl.ds(base + off, sz)], dbufs[b], semi)

    def wait_idx(sx, dbufs, semi):
        pltpu.make_async_copy(src_hbm.at[pl.ds(e0, _GK)], sx, semi).wait()
        for b, (off, sz) in enumerate(_CH):
            pltpu.make_async_copy(dst_hbm.at[pl.ds(e0, sz)], dbufs[b], semi).wait()

    def proc(sx, dbufs, semi):
        wait_idx(sx, dbufs, semi)
        gd = [pltpu.async_copy(x_hbm.at[sx.at[pl.ds(off, sz)]],
                               rows.at[pl.ds(off, sz)], sgs[b])
              for b, (off, sz) in enumerate(_CH)]
        sd = []
        for b, (off, sz) in enumerate(_CH):
            gd[b].wait()
            sd.append(pltpu.async_copy(rows.at[pl.ds(off, sz)],
                                       acc.at[dbufs[b]], sems, add=True))
        for d in sd:
            d.wait()

    fire_idx(e0, sxa, abufs, semia)

    def pair(t, carry):
        base_a = e0 + (2 * t) * _GK
        base_b = base_a + _GK
        base_n = jnp.where(t < _GROUPS // 2 - 1, base_a + 2 * _GK, e0)
        fire_idx(base_b, sxb, bbufs, semib)
        proc(sxa, abufs, semia)
        fire_idx(base_n, sxa, abufs, semia)
        proc(sxb, bbufs, semib)
        return carry

    lax.fori_loop(0, _GROUPS // 2, pair, 0)
    wait_idx(sxa, abufs, semia)
    plsc.subcore_barrier()
    pltpu.sync_copy(acc.at[pl.ds(r0, _RPT)], part_hbm.at[c, pl.ds(r0, _RPT)])


_seg_sum = pl.kernel(
    _seg_body,
    out_type=jax.ShapeDtypeStruct((_NC, _NP, _D), jnp.float32),
    mesh=_mesh,
    scratch_types=[pltpu.VMEM_SHARED((_NP, _D), jnp.float32)]
                  + [pltpu.VMEM((_GK,), jnp.int32)]
                  + [pltpu.VMEM((sz,), jnp.int32) for (_o, sz) in _CH]
                  + [pltpu.VMEM((_GK,), jnp.int32)]
                  + [pltpu.VMEM((sz,), jnp.int32) for (_o, sz) in _CH]
                  + [pltpu.VMEM((_GK, _D), jnp.float32)]
                  + [pltpu.SemaphoreType.DMA] * (2 + _NB)
                  + [pltpu.SemaphoreType.DMA],
)


def _deg_body(dst_hbm, zrow_hbm, onesr_hbm, part_hbm, *scr):
    acc = scr[0]
    abufs = scr[1:1 + _NB]
    bbufs = scr[1 + _NB:1 + 2 * _NB]
    onesv = scr[1 + 2 * _NB]
    semia, semib, sems = scr[2 + 2 * _NB], scr[3 + 2 * _NB], scr[4 + 2 * _NB]
    c = lax.axis_index("c")
    s = lax.axis_index("s")
    wid = c * _NS + s
    r0 = s * _RPT
    pltpu.sync_copy(zrow_hbm.at[pl.ds(r0, _RPT)], acc.at[pl.ds(r0, _RPT)])
    pltpu.sync_copy(onesr_hbm, onesv)
    plsc.subcore_barrier()
    e0 = wid * _EPT

    def fire_idx(base, dbufs, semi):
        for b, (off, sz) in enumerate(_CH):
            pltpu.async_copy(dst_hbm.at[pl.ds(base + off, sz)], dbufs[b], semi)

    def wait_idx(dbufs, semi):
        for b, (off, sz) in enumerate(_CH):
            pltpu.make_async_copy(dst_hbm.at[pl.ds(e0, sz)], dbufs[b], semi).wait()

    def proc(dbufs, semi):
        wait_idx(dbufs, semi)
        sd = [pltpu.async_copy(onesv.at[pl.ds(0, sz)], acc.at[dbufs[b]], sems, add=True)
              for b, (off, sz) in enumerate(_CH)]
        for d in sd:
            d.wait()

    fire_idx(e0, abufs, semia)

    def pair(t, carry):
        base_a = e0 + (2 * t) * _GK
        base_b = base_a + _GK
        base_n = jnp.where(t < _GROUPS // 2 - 1, base_a + 2 * _GK, e0)
        fire_idx(base_b, bbufs, semib)
        proc(abufs, semia)
        fire_idx(base_n, abufs, semia)
        proc(bbufs, semib)
        return carry

    lax.fori_loop(0, _GROUPS // 2, pair, 0)
    wait_idx(abufs, semia)
    plsc.subcore_barrier()
    pltpu.sync_copy(acc.at[pl.ds(r0, _RPT)], part_hbm.at[c, pl.ds(r0, _RPT)])


_deg_sum = pl.kernel(
    _deg_body,
    out_type=jax.ShapeDtypeStruct((_NC, _NP, _D), jnp.float32),
    mesh=_mesh,
    scratch_types=[pltpu.VMEM_SHARED((_NP, _D), jnp.float32)]
                  + [pltpu.VMEM((sz,), jnp.int32) for (_o, sz) in _CH]
                  + [pltpu.VMEM((sz,), jnp.int32) for (_o, sz) in _CH]
                  + [pltpu.VMEM((_CH[0][1], _D), jnp.float32)]
                  + [pltpu.SemaphoreType.DMA] * 3,
)


def _matmul_t(a, w):
    return lax.dot_general(a, w, (((1,), (1,)), ((), ())),
                           preferred_element_type=jnp.float32)


def _leaky(v, slope):
    return jnp.where(v > 0, v, slope * v)


def _graphnorm(h, ind, rcnt, gamma, beta, alpha):
    mean = lax.dot_general(ind, h, (((0,), (0,)), ((), ())),
                           preferred_element_type=jnp.float32) * rcnt
    sub = h - alpha * lax.dot_general(ind, mean, (((1,), (0,)), ((), ())),
                                      preferred_element_type=jnp.float32)
    var = lax.dot_general(ind, sub * sub, (((0,), (0,)), ((), ())),
                          preferred_element_type=jnp.float32) * rcnt
    inv = lax.rsqrt(var + 1e-5)
    return gamma * sub * lax.dot_general(ind, inv, (((1,), (0,)), ((), ())),
                                         preferred_element_type=jnp.float32) + beta


def _ind_rcnt(batch2d):
    ind = (lax.broadcasted_iota(jnp.int32, (_N, _B), 1) == batch2d)
    ind = ind.astype(jnp.float32)
    cnt = lax.dot_general(ind, jnp.ones((_N, 1), jnp.float32),
                          (((0,), (0,)), ((), ())),
                          preferred_element_type=jnp.float32)
    rcnt = 1.0 / jnp.maximum(cnt, 1.0)
    return ind, rcnt


def _tc1_body(x_ref, part_ref, degp_ref, batch_ref, wl_ref, bl_ref, wr_ref,
              g_ref, be_ref, a_ref, h1_ref):
    ssum = part_ref[0, :_N] + part_ref[1, :_N]
    deg = degp_ref[0, :_N] + degp_ref[1, :_N]
    agg = ssum / jnp.maximum(deg, 1.0)
    pre = _matmul_t(agg, wl_ref[...]) + bl_ref[...] + _matmul_t(x_ref[...], wr_ref[...])
    h = _leaky(pre, 0.01)
    ind, rcnt = _ind_rcnt(batch_ref[...])
    h1_ref[...] = _graphnorm(h, ind, rcnt, g_ref[...], be_ref[...], a_ref[...])


def _tc2_body(h1_ref, part_ref, degp_ref, batch_ref, wl_ref, bl_ref, wr_ref,
              g_ref, be_ref, a_ref, wc1_ref, bc1_ref, bng_ref, bnb_ref,
              bnm_ref, bnv_ref, wc2_ref, bc2_ref, out_ref):
    h1 = h1_ref[...]
    ssum = part_ref[0, :_N] + part_ref[1, :_N]
    deg = degp_ref[0, :_N] + degp_ref[1, :_N]
    agg = ssum / jnp.maximum(deg, 1.0)
    pre = _matmul_t(agg, wl_ref[...]) + bl_ref[...] + _matmul_t(h1, wr_ref[...])
    r = _leaky(pre, 0.01)
    ind, rcnt = _ind_rcnt(batch_ref[...])
    r = _graphnorm(r, ind, rcnt, g_ref[...], be_ref[...], a_ref[...])
    h = h1 + r
    p = lax.dot_general(ind, h, (((0,), (0,)), ((), ())),
                        preferred_element_type=jnp.float32) * rcnt
    c1 = _matmul_t(p, wc1_ref[...]) + bc1_ref[...]
    c1 = (c1 - bnm_ref[...]) * lax.rsqrt(bnv_ref[...] + 1e-5) * bng_ref[...] + bnb_ref[...]
    c1 = _leaky(c1, 0.1)
    c2 = _matmul_t(c1, wc2_ref[...]) + bc2_ref[...]
    m = jnp.max(c2, axis=1, keepdims=True)
    sh = c2 - m
    out_ref[...] = sh - jnp.log(jnp.sum(jnp.exp(sh), axis=1, keepdims=True))


_tc1 = pl.pallas_call(_tc1_body, out_shape=jax.ShapeDtypeStruct((_N, _D), jnp.float32))
_tc2 = pl.pallas_call(_tc2_body, out_shape=jax.ShapeDtypeStruct((_B, 2), jnp.float32))


def kernel(x, edge_index, batch, Wl1, bl1, Wr1, g1, be1, a1, Wl2, bl2, Wr2,
           g2, be2, a2, Wc1, bc1, bng, bnb, bnm, bnv, Wc2, bc2):
    src = edge_index[0]
    dst = edge_index[1]
    zrow = jnp.zeros((_NP, _D), jnp.float32)
    onesr = jnp.ones((_CH[0][1], _D), jnp.float32)
    batch2d = batch.reshape(_N, 1)

    part1 = _seg_sum(x, src, dst, zrow)
    degp = _deg_sum(dst, zrow, onesr)
    h1 = _tc1(x, part1, degp, batch2d, Wl1, bl1.reshape(1, _D), Wr1,
              g1.reshape(1, _D), be1.reshape(1, _D), a1.reshape(1, _D))
    part2 = _seg_sum(h1, src, dst, zrow)
    out = _tc2(h1, part2, degp, batch2d, Wl2, bl2.reshape(1, _D), Wr2,
               g2.reshape(1, _D), be2.reshape(1, _D), a2.reshape(1, _D),
               Wc1, bc1.reshape(1, _B), bng.reshape(1, _B), bnb.reshape(1, _B),
               bnm.reshape(1, _B), bnv.reshape(1, _B), Wc2, bc2.reshape(1, 2))
    return out
